# Initial kernel scaffold; baseline (speedup 1.0000x reference)
import jax
import jax.numpy as jnp
from jax.experimental import pallas as pl


def kernel(Fa, Fb, Wq, Wk, Wv, Wp, a_idx, b_idx):
    raise NotImplementedError("write your pallas kernel here")


# speedup vs baseline: 41.2879x; 41.2879x over previous
import functools
import math

import jax
import jax.numpy as jnp
from jax import lax
from jax.experimental import pallas as pl
from jax.experimental.pallas import tpu as pltpu
from jax.experimental.pallas import tpu_sc as plsc

NA = 10000
NB = 10000
E = 160000
D = 256
H = 8
DH = D // H
NSUB = 32
RPS = 320
ROWS_BUF = 320
EB = 64
NVR = D // 16
HVR = DH // 16
SCALE = 1.0 / math.sqrt(DH)
NEG = -1e30


def _matmul_qkv(Fa, Fb, Wq, Wk, Wv):
    bm = 1000
    g = NA // bm

    def body(fa, fb, wq, wk, wv, q, k, v):
        q[...] = jnp.dot(fa[...], wq[...], preferred_element_type=jnp.float32)
        k[...] = jnp.dot(fb[...], wk[...], preferred_element_type=jnp.float32)
        v[...] = jnp.dot(fb[...], wv[...], preferred_element_type=jnp.float32)

    return pl.pallas_call(
        body,
        grid=(g,),
        in_specs=[
            pl.BlockSpec((bm, Fa.shape[1]), lambda i: (i, 0)),
            pl.BlockSpec((bm, Fb.shape[1]), lambda i: (i, 0)),
            pl.BlockSpec(Wq.shape, lambda i: (0, 0)),
            pl.BlockSpec(Wk.shape, lambda i: (0, 0)),
            pl.BlockSpec(Wv.shape, lambda i: (0, 0)),
        ],
        out_specs=[
            pl.BlockSpec((bm, D), lambda i: (i, 0)),
            pl.BlockSpec((bm, D), lambda i: (i, 0)),
            pl.BlockSpec((bm, D), lambda i: (i, 0)),
        ],
        out_shape=[
            jax.ShapeDtypeStruct((NA, D), jnp.float32),
            jax.ShapeDtypeStruct((NB, D), jnp.float32),
            jax.ShapeDtypeStruct((NB, D), jnp.float32),
        ],
    )(Fa, Fb, Wq, Wk, Wv)


def _matmul_final(Fa, att, Wp):
    bm = 1000
    g = NA // bm

    def body(fa, at, wp, o):
        o[...] = fa[...] + jnp.dot(at[...], wp[...],
                                   preferred_element_type=jnp.float32)

    return pl.pallas_call(
        body,
        grid=(g,),
        in_specs=[
            pl.BlockSpec((bm, Fa.shape[1]), lambda i: (i, 0)),
            pl.BlockSpec((bm, D), lambda i: (i, 0)),
            pl.BlockSpec(Wp.shape, lambda i: (0, 0)),
        ],
        out_specs=pl.BlockSpec((bm, Fa.shape[1]), lambda i: (i, 0)),
        out_shape=jax.ShapeDtypeStruct((NA, Fa.shape[1]), jnp.float32),
    )(Fa, att, Wp)


def _sc_edge(Q, K, V, a_pad, b_pad, bounds):
    E_pad = a_pad.shape[0]
    mesh = plsc.VectorSubcoreMesh(core_axis_name="c", subcore_axis_name="s")

    @functools.partial(
        pl.kernel,
        mesh=mesh,
        compiler_params=pltpu.CompilerParams(needs_layout_passes=False),
        out_type=[
            jax.ShapeDtypeStruct((NA, D), jnp.float32),
            jax.ShapeDtypeStruct((E_pad * 16,), jnp.float32),
        ],
        scratch_types=[
            pltpu.VMEM((48,), jnp.int32),
            pltpu.VMEM((EB + 16,), jnp.int32),
            pltpu.VMEM((EB,), jnp.int32),
            pltpu.VMEM((EB,), jnp.int32),
            pltpu.VMEM((EB, D), jnp.float32),
            pltpu.VMEM((EB, D), jnp.float32),
            pltpu.VMEM((EB * 16,), jnp.float32),
            pltpu.VMEM((ROWS_BUF * 16,), jnp.float32),
            pltpu.VMEM((ROWS_BUF * 16,), jnp.float32),
            pltpu.VMEM((ROWS_BUF, D), jnp.float32),
            pltpu.SemaphoreType.DMA,
            pltpu.SemaphoreType.DMA,
        ],
    )
    def sc_kernel(q_hbm, k_hbm, v_hbm, aidx_hbm, bidx_hbm, bounds_hbm,
                  out_hbm, lg_hbm,
                  bounds_v, aidx_s, aidx_g, bidx_v, qbuf, kvbuf, lbuf,
                  mbuf, dbuf, rowsbuf, sem0, sem1):
        def sv(ref, i):
            return ref[pl.ds(i, 16)][0]

        wid = lax.axis_index("s") * 2 + lax.axis_index("c")
        pltpu.sync_copy(bounds_hbm, bounds_v)
        e_lo = sv(bounds_v, wid)
        e_hi = sv(bounds_v, wid + 1)
        a0 = wid * RPS
        a1 = jnp.minimum(a0 + RPS, NA)
        base0 = (e_lo // 8) * 8
        nblk = (e_hi - base0 + EB - 1) // EB
        li = lax.iota(jnp.int32, 16)
        zero16 = jnp.zeros((16,), jnp.float32)
        negvec = jnp.full((16,), NEG, jnp.float32)

        def zrow(r, c):
            for j in range(NVR):
                rowsbuf[r, pl.ds(16 * j, 16)] = zero16
            return c

        lax.fori_loop(0, ROWS_BUF, zrow, 0)

        def dot_qk(e2):
            ps = []
            for j in range(NVR):
                qj = qbuf[e2, pl.ds(16 * j, 16)]
                kj = kvbuf[e2, pl.ds(16 * j, 16)]
                ps.append(qj * kj)
            l = zero16
            for h in range(H):
                w = ps[HVR * h]
                for t in range(1, HVR):
                    w = w + ps[HVR * h + t]
                s = jnp.sum(w)
                l = jnp.where(li == h, s, l)
            return l * SCALE

        def bcast_head(w, h):
            s = jnp.sum(jnp.where(li == h, w, 0.0))
            return jnp.full((16,), s, jnp.float32)

        def blkA(blk, carry):
            cur_ia, m = carry
            base = base0 + blk * EB
            pltpu.sync_copy(aidx_hbm.at[pl.ds(base, EB + 16)], aidx_s)
            pltpu.sync_copy(aidx_hbm.at[pl.ds(base, EB)], aidx_g)
            pltpu.sync_copy(bidx_hbm.at[pl.ds(base, EB)], bidx_v)
            cq = pltpu.async_copy(q_hbm.at[aidx_g], qbuf, sem0)
            ck = pltpu.async_copy(k_hbm.at[bidx_v], kvbuf, sem1)
            cq.wait()
            ck.wait()

            def edge(e2, c):
                cur_ia, m = c
                ge = base + e2
                valid = (ge >= e_lo) & (ge < e_hi)
                ae = sv(aidx_s, e2)
                ia = ae - a0
                l = dot_qk(e2)
                lbuf[pl.ds(e2 * 16, 16)] = l
                changed = valid & (ia != cur_ia)

                @pl.when(changed & (cur_ia >= 0))
                def _():
                    mbuf[pl.ds(cur_ia * 16, 16)] = m

                m = jnp.where(changed, negvec, m)
                m = jnp.where(valid, jnp.maximum(m, l), m)
                cur_ia = jnp.where(valid, ia, cur_ia)
                return cur_ia, m

            cur_ia, m = lax.fori_loop(0, EB, edge, (cur_ia, m))
            pltpu.sync_copy(lbuf, lg_hbm.at[pl.ds(base * 16, EB * 16)])
            return cur_ia, m

        cur_ia, m = lax.fori_loop(0, nblk, blkA, (jnp.int32(-1), negvec))

        @pl.when(cur_ia >= 0)
        def _():
            mbuf[pl.ds(cur_ia * 16, 16)] = m

        def blkB(blk, carry):
            cur_ia, d, mrow = carry
            base = base0 + blk * EB
            pltpu.sync_copy(aidx_hbm.at[pl.ds(base, EB + 16)], aidx_s)
            pltpu.sync_copy(lg_hbm.at[pl.ds(base * 16, EB * 16)], lbuf)

            def edge(e2, c):
                cur_ia, d, mrow = c
                ge = base + e2
                valid = (ge >= e_lo) & (ge < e_hi)
                ae = sv(aidx_s, e2)
                ia = ae - a0
                iac = jnp.clip(ia, 0, ROWS_BUF - 1)
                l = lbuf[pl.ds(e2 * 16, 16)]
                changed = valid & (ia != cur_ia)

                @pl.when(changed & (cur_ia >= 0))
                def _():
                    dbuf[pl.ds(cur_ia * 16, 16)] = d

                mrow = jnp.where(changed, mbuf[pl.ds(iac * 16, 16)], mrow)
                d = jnp.where(changed, zero16, d)
                d = jnp.where(valid, d + jnp.exp(l - mrow), d)
                cur_ia = jnp.where(valid, ia, cur_ia)
                return cur_ia, d, mrow

            cur_ia, d, mrow = lax.fori_loop(0, EB, edge, (cur_ia, d, mrow))
            return cur_ia, d, mrow

        cur_ia, d, _ = lax.fori_loop(0, nblk, blkB,
                                     (jnp.int32(-1), zero16, zero16))

        @pl.when(cur_ia >= 0)
        def _():
            dbuf[pl.ds(cur_ia * 16, 16)] = d

        def blkC(blk, carry):
            cur_ia, mrow, dinv, acc = carry
            base = base0 + blk * EB
            pltpu.sync_copy(aidx_hbm.at[pl.ds(base, EB + 16)], aidx_s)
            pltpu.sync_copy(bidx_hbm.at[pl.ds(base, EB)], bidx_v)
            pltpu.sync_copy(lg_hbm.at[pl.ds(base * 16, EB * 16)], lbuf)
            cv = pltpu.async_copy(v_hbm.at[bidx_v], kvbuf, sem1)
            cv.wait()

            def edge(e2, c):
                cur_ia, mrow, dinv, acc = c
                ge = base + e2
                valid = (ge >= e_lo) & (ge < e_hi)
                ae = sv(aidx_s, e2)
                ia = ae - a0
                iac = jnp.clip(ia, 0, ROWS_BUF - 1)
                l = lbuf[pl.ds(e2 * 16, 16)]
                changed = valid & (ia != cur_ia)

                @pl.when(changed & (cur_ia >= 0))
                def _():
                    for j in range(NVR):
                        rowsbuf[cur_ia, pl.ds(16 * j, 16)] = acc[j]

                mrow = jnp.where(changed, mbuf[pl.ds(iac * 16, 16)], mrow)
                dinv = jnp.where(changed, 1.0 / dbuf[pl.ds(iac * 16, 16)], dinv)
                w = jnp.exp(l - mrow) * dinv
                bw = [bcast_head(w, h) for h in range(H)]
                newacc = []
                for j in range(NVR):
                    aj = jnp.where(changed, zero16, acc[j])
                    vj = kvbuf[e2, pl.ds(16 * j, 16)]
                    newacc.append(
                        jnp.where(valid, aj + bw[j // HVR] * vj, aj))
                cur_ia = jnp.where(valid, ia, cur_ia)
                return cur_ia, mrow, dinv, tuple(newacc)

            return lax.fori_loop(0, EB, edge, (cur_ia, mrow, dinv, acc))

        acc0 = tuple(zero16 for _ in range(NVR))
        cur_ia, _, _, acc = lax.fori_loop(
            0, nblk, blkC, (jnp.int32(-1), zero16, zero16, acc0))

        @pl.when(cur_ia >= 0)
        def _():
            for j in range(NVR):
                rowsbuf[cur_ia, pl.ds(16 * j, 16)] = acc[j]

        for kk in range(ROWS_BUF // 64):
            s = jnp.minimum(a0 + 64 * kk, a1 - 64)
            pltpu.sync_copy(rowsbuf.at[pl.ds(s - a0, 64)],
                            out_hbm.at[pl.ds(s, 64)])

    return sc_kernel(Q, K, V, a_pad, b_pad, bounds)


def kernel(Fa, Fb, Wq, Wk, Wv, Wp, a_idx, b_idx):
    a32 = a_idx.astype(jnp.int32)
    b32 = b_idx.astype(jnp.int32)
    Q, K, V = _matmul_qkv(Fa, Fb, Wq, Wk, Wv)
    ladder = jnp.minimum(jnp.arange(NSUB + 1, dtype=jnp.int32) * RPS, NA)
    bounds = jnp.searchsorted(a32, ladder, side="left").astype(jnp.int32)
    bounds = jnp.pad(bounds, (0, 48 - NSUB - 1))
    a_pad = jnp.pad(a32, (0, 2 * EB))
    b_pad = jnp.pad(b32, (0, 2 * EB))
    att, _ = _sc_edge(Q, K, V, a_pad, b_pad, bounds)
    return _matmul_final(Fa, att, Wp)
